# baseline (speedup 1.0000x reference)
import functools

import jax
import jax.numpy as jnp
from jax import lax
from jax.experimental import pallas as pl
from jax.experimental.pallas import tpu as pltpu
from jax.experimental.pallas import tpu_sc as plsc

VOCAB = 1000000
L_CTX = 200
D = 32
BATCH = 4096

NC = 2
NS = 16
NW = NC * NS

N = BATCH * L_CTX
R_PER_W = N // NW
C = 1600
N_CHUNKS = R_PER_W // C


def _make_kernel():
    mesh = plsc.VectorSubcoreMesh(
        core_axis_name="c", subcore_axis_name="s",
        num_cores=NC, num_subcores=NS)

    @functools.partial(
        pl.kernel,
        out_type=jax.ShapeDtypeStruct((N, D), jnp.float32),
        mesh=mesh,
        scratch_types=[
            pltpu.VMEM((C,), jnp.int32),
            pltpu.VMEM((C, D), jnp.float32),
            pltpu.SemaphoreType.DMA,
        ],
        compiler_params=pltpu.CompilerParams(use_tc_tiling_on_sc=False),
    )
    def gather_kernel(x_hbm, tok_hbm, out_hbm, idx_v, rows_v, sem):
        wid = lax.axis_index("s") * NC + lax.axis_index("c")
        base = wid * R_PER_W

        def chunk_body(ci, _):
            cb = base + ci * C
            pltpu.sync_copy(x_hbm.at[pl.ds(cb, C)], idx_v)
            pltpu.async_copy(tok_hbm.at[idx_v], rows_v, sem).wait()
            pltpu.sync_copy(rows_v, out_hbm.at[pl.ds(cb, C)])
            return 0

        lax.fori_loop(0, N_CHUNKS, chunk_body, 0)

    return gather_kernel


_gather_kernel = _make_kernel()


@jax.jit
def kernel(x, token_table, pos_table):
    x_flat = x.reshape(N).astype(jnp.int32)
    rows = _gather_kernel(x_flat, token_table)
    return rows.reshape(BATCH, L_CTX, D) + pos_table[None, :, :]


# speedup vs baseline: 1.0029x; 1.0029x over previous
import functools

import jax
import jax.numpy as jnp
from jax import lax
from jax.experimental import pallas as pl
from jax.experimental.pallas import tpu as pltpu
from jax.experimental.pallas import tpu_sc as plsc

VOCAB = 1000000
L_CTX = 200
D = 32
BATCH = 4096

NC = 2
NS = 16
NW = NC * NS

N = BATCH * L_CTX
R_PER_W = N // NW
C = 1600
N_CHUNKS = R_PER_W // C


def _make_kernel():
    mesh = plsc.VectorSubcoreMesh(
        core_axis_name="c", subcore_axis_name="s",
        num_cores=NC, num_subcores=NS)

    @functools.partial(
        pl.kernel,
        out_type=jax.ShapeDtypeStruct((N, D), jnp.float32),
        mesh=mesh,
        scratch_types=[
            pltpu.VMEM((C,), jnp.int32),
            pltpu.VMEM((C, D), jnp.float32),
            pltpu.SemaphoreType.DMA,
        ],
        compiler_params=pltpu.CompilerParams(use_tc_tiling_on_sc=False),
    )
    def gather_kernel(x_hbm, tok_hbm, out_hbm, idx_v, rows_v, sem):
        wid = lax.axis_index("s") * NC + lax.axis_index("c")
        base = wid * R_PER_W

        def chunk_body(ci, _):
            cb = base + ci * C
            pltpu.sync_copy(x_hbm.at[pl.ds(cb, C)], idx_v)
            pltpu.async_copy(tok_hbm.at[idx_v], rows_v, sem).wait()
            pltpu.sync_copy(rows_v, out_hbm.at[pl.ds(cb, C)])
            return 0

        lax.fori_loop(0, N_CHUNKS, chunk_body, 0)

    return gather_kernel


_gather_kernel = _make_kernel()


@jax.jit
def kernel(x, token_table, pos_table):
    x_flat = x.reshape(N).astype(jnp.int32)
    tok_flat = lax.optimization_barrier(token_table.reshape(VOCAB * D))
    tok_lin = tok_flat.reshape(VOCAB, D)
    rows = _gather_kernel(x_flat, tok_lin)
    return rows.reshape(BATCH, L_CTX, D) + pos_table[None, :, :]
